# baseline (speedup 1.0000x reference)
import jax
import jax.numpy as jnp
from jax.experimental import pallas as pl
from jax.experimental.pallas import tpu as pltpu

_S = 392
_INV_S = 1.0 / _S


def _rowmean_kernel(x_ref, o_ref):
    x = x_ref[...]
    folded = x[:, 0:128] + x[:, 128:256] + x[:, 256:384]
    total = (jnp.sum(folded, axis=-1, keepdims=True)
             + jnp.sum(x[:, 384:392], axis=-1, keepdims=True))
    o_ref[...] = total * _INV_S


def _rowmean(x2d, *, tr):
    rows, s = x2d.shape
    grid = (rows // tr,)
    return pl.pallas_call(
        _rowmean_kernel,
        out_shape=jax.ShapeDtypeStruct((rows, 1), x2d.dtype),
        grid=grid,
        in_specs=[pl.BlockSpec((tr, s), lambda i: (i, 0))],
        out_specs=pl.BlockSpec((tr, 1), lambda i: (i, 0)),
        compiler_params=pltpu.CompilerParams(
            dimension_semantics=("parallel",)),
    )(x2d)


def kernel(x2d):
    return _rowmean(x2d, tr=1024)


# speedup vs baseline: 1.0932x; 1.0932x over previous
import jax
import jax.numpy as jnp
from jax.experimental import pallas as pl
from jax.experimental.pallas import tpu as pltpu

_S = 392
_INV_S = 1.0 / _S


def _rowmean_kernel(x_ref, o_ref):
    x = x_ref[...]
    folded = x[:, 0:128] + x[:, 128:256] + x[:, 256:384]
    total = (jnp.sum(folded, axis=-1, keepdims=True)
             + jnp.sum(x[:, 384:392], axis=-1, keepdims=True))
    o_ref[...] = total * _INV_S


def _rowmean(x2d, *, tr):
    rows, s = x2d.shape
    grid = (rows // tr,)
    return pl.pallas_call(
        _rowmean_kernel,
        out_shape=jax.ShapeDtypeStruct((rows, 1), x2d.dtype),
        grid=grid,
        in_specs=[pl.BlockSpec((tr, s), lambda i: (i, 0))],
        out_specs=pl.BlockSpec((tr, 1), lambda i: (i, 0)),
        compiler_params=pltpu.CompilerParams(
            dimension_semantics=("parallel",)),
    )(x2d)


def kernel(x2d):
    return _rowmean(x2d, tr=4096)


# speedup vs baseline: 1.3228x; 1.2101x over previous
import functools

import jax
import jax.numpy as jnp
from jax.experimental import pallas as pl
from jax.experimental.pallas import tpu as pltpu

_S = 392
_INV_S = 1.0 / _S


def _rowmean_mxu_kernel(x_ref, o_ref, *, tr):
    x = x_ref[...]
    ones = jnp.ones((1, x.shape[1]), jnp.float32)
    s = jax.lax.dot_general(ones, x, (((1,), (1,)), ((), ())),
                            preferred_element_type=jnp.float32)
    o_ref[...] = s.reshape(tr // 128, 128) * _INV_S


def _rowmean_vpu_kernel(x_ref, o_ref, *, tr):
    x = x_ref[...]
    folded = x[:, 0:128] + x[:, 128:256] + x[:, 256:384]
    total = (jnp.sum(folded, axis=-1, keepdims=True)
             + jnp.sum(x[:, 384:392], axis=-1, keepdims=True)) * _INV_S
    o_ref[...] = total.reshape(tr // 128, 128)


_KERNELS = {"mxu": _rowmean_mxu_kernel, "vpu": _rowmean_vpu_kernel}


def _rowmean(x2d, *, tr, body):
    rows, s = x2d.shape
    grid = (rows // tr,)
    dense = pl.pallas_call(
        functools.partial(_KERNELS[body], tr=tr),
        out_shape=jax.ShapeDtypeStruct((rows // 128, 128), x2d.dtype),
        grid=grid,
        in_specs=[pl.BlockSpec((tr, s), lambda i: (i, 0))],
        out_specs=pl.BlockSpec((tr // 128, 128), lambda i: (i, 0)),
        compiler_params=pltpu.CompilerParams(
            dimension_semantics=("parallel",)),
    )(x2d)
    return dense.reshape(rows, 1)


def kernel(x2d):
    return _rowmean(x2d, tr=4096, body="mxu")


# speedup vs baseline: 1.3271x; 1.0032x over previous
import functools

import jax
import jax.numpy as jnp
from jax.experimental import pallas as pl
from jax.experimental.pallas import tpu as pltpu

_S = 392
_INV_S = 1.0 / _S


def _rowmean_mxu_kernel(x_ref, o_ref, *, tr):
    x = x_ref[...]
    ones = jnp.ones((1, x.shape[1]), jnp.float32)
    s = jax.lax.dot_general(ones, x, (((1,), (1,)), ((), ())),
                            preferred_element_type=jnp.float32)
    o_ref[...] = s.reshape(tr // 128, 128) * _INV_S


def _rowmean_vpu_kernel(x_ref, o_ref, *, tr):
    x = x_ref[...]
    folded = x[:, 0:128] + x[:, 128:256] + x[:, 256:384]
    total = (jnp.sum(folded, axis=-1, keepdims=True)
             + jnp.sum(x[:, 384:392], axis=-1, keepdims=True)) * _INV_S
    o_ref[...] = total.reshape(tr // 128, 128)


_KERNELS = {"mxu": _rowmean_mxu_kernel, "vpu": _rowmean_vpu_kernel}


def _rowmean(x2d, *, tr, body):
    rows, s = x2d.shape
    grid = (rows // tr,)
    dense = pl.pallas_call(
        functools.partial(_KERNELS[body], tr=tr),
        out_shape=jax.ShapeDtypeStruct((rows // 128, 128), x2d.dtype),
        grid=grid,
        in_specs=[pl.BlockSpec((tr, s), lambda i: (i, 0))],
        out_specs=pl.BlockSpec((tr // 128, 128), lambda i: (i, 0)),
        compiler_params=pltpu.CompilerParams(
            dimension_semantics=("parallel",)),
    )(x2d)
    return dense.reshape(rows, 1)


def kernel(x2d):
    return _rowmean(x2d, tr=4096, body="vpu")
